# baseline (device time: 19751 ns/iter reference)
import jax
import jax.numpy as jnp
from jax import lax
from jax.experimental import pallas as pl
from jax.experimental.pallas import tpu as pltpu

B, H, D, BS = 8, 8, 64, 16
NB = 64
SCALE = D ** -0.5


def kernel(Q, K, V, bt, lens):
    n_loc, bs, h, d = K.shape
    n_keys = n_loc * bs

    def body(q_ref, k_ref, v_ref, bt_ref, lens_ref, out_ref,
             o_send, m_send, d_send, o_recv, m_recv, d_recv,
             send_sems, recv_sems):
        my_x = lax.axis_index("x")
        my_y = lax.axis_index("y")
        my_z = lax.axis_index("z")
        peer = (1 - my_x, my_y, my_z)

        bt3 = bt_ref[...][:, :, None]
        key_page = (lax.broadcasted_iota(jnp.int32, (B, NB, n_keys), 2) // bs
                    + my_x * n_loc)
        j_idx = lax.broadcasted_iota(jnp.int32, (B, NB, n_keys), 1)
        lens3 = lens_ref[...][:, :, None]
        hit = (bt3 == key_page) & (j_idx < lens3)
        w = jnp.sum(jnp.where(hit, 1.0, 0.0), axis=1)

        os_, ms_, ds_ = [], [], []
        for hh in range(h):
            qh = q_ref[:, 0, hh, :]
            kh = k_ref[:, :, hh, :].reshape(n_keys, d)
            s = lax.dot_general(qh, kh, (((1,), (1,)), ((), ())),
                                preferred_element_type=jnp.float32) * SCALE
            s = jnp.where(w > 0, s, -1e30)
            mh = jnp.max(s, axis=1)
            p = jnp.exp(s - mh[:, None]) * w
            dh = jnp.sum(p, axis=1)
            vh = v_ref[:, :, hh, :].reshape(n_keys, d)
            oh = lax.dot_general(p, vh, (((1,), (0,)), ((), ())),
                                 preferred_element_type=jnp.float32)
            os_.append(oh[:, None, :])
            ms_.append(mh[:, None])
            ds_.append(dh[:, None])
        o_loc = jnp.concatenate(os_, axis=1)
        m_loc = jnp.concatenate(ms_, axis=1)
        d_loc = jnp.concatenate(ds_, axis=1)

        o_send[...] = o_loc
        m_send[...] = m_loc
        d_send[...] = d_loc

        barrier = pltpu.get_barrier_semaphore()
        pl.semaphore_signal(barrier, inc=1, device_id=peer,
                            device_id_type=pl.DeviceIdType.MESH)
        pl.semaphore_wait(barrier, 1)

        copies = []
        for i, (src, dst) in enumerate(
                ((o_send, o_recv), (m_send, m_recv), (d_send, d_recv))):
            c = pltpu.make_async_remote_copy(
                src_ref=src, dst_ref=dst,
                send_sem=send_sems.at[i], recv_sem=recv_sems.at[i],
                device_id=peer, device_id_type=pl.DeviceIdType.MESH)
            c.start()
            copies.append(c)
        for c in copies:
            c.wait()

        m_rem = m_recv[...]
        d_rem = d_recv[...]
        o_rem = o_recv[...]
        mm = jnp.maximum(m_loc, m_rem)
        a_loc = jnp.exp(m_loc - mm)
        a_rem = jnp.exp(m_rem - mm)
        num = o_loc * a_loc[:, :, None] + o_rem * a_rem[:, :, None]
        den = d_loc * a_loc + d_rem * a_rem
        out_ref[:, 0, :, :] = num / den[:, :, None]

    lens2 = lens.reshape(B, 1)
    return pl.pallas_call(
        body,
        out_shape=jax.ShapeDtypeStruct(Q.shape, jnp.float32),
        in_specs=[pl.BlockSpec(memory_space=pltpu.VMEM)] * 5,
        out_specs=pl.BlockSpec(memory_space=pltpu.VMEM),
        scratch_shapes=[
            pltpu.VMEM((B, h, d), jnp.float32),
            pltpu.VMEM((B, h), jnp.float32),
            pltpu.VMEM((B, h), jnp.float32),
            pltpu.VMEM((B, h, d), jnp.float32),
            pltpu.VMEM((B, h), jnp.float32),
            pltpu.VMEM((B, h), jnp.float32),
            pltpu.SemaphoreType.DMA((3,)),
            pltpu.SemaphoreType.DMA((3,)),
        ],
        compiler_params=pltpu.CompilerParams(collective_id=0),
    )(Q, K, V, bt, lens2)


# device time: 11802 ns/iter; 1.6735x vs baseline; 1.6735x over previous
import jax
import jax.numpy as jnp
from jax import lax
from jax.experimental import pallas as pl
from jax.experimental.pallas import tpu as pltpu

B, H, D, BS = 8, 8, 64, 16
NB = 64
SCALE = D ** -0.5


def kernel(Q, K, V, bt, lens):
    n_loc, bs, h, d = K.shape
    n_keys = n_loc * bs
    R = B * h
    C = h * d

    def body(q_ref, k_ref, v_ref, bt_ref, lens_ref, out_ref,
             send_buf, recv_buf, send_sem, recv_sem):
        my_x = lax.axis_index("x")
        my_y = lax.axis_index("y")
        my_z = lax.axis_index("z")
        peer = (1 - my_x, my_y, my_z)

        j2 = lax.broadcasted_iota(jnp.int32, (B, NB), 1)
        btm = jnp.where(j2 < lens_ref[...], bt_ref[...], -1)
        pg = (lax.broadcasted_iota(jnp.int32, (B, NB, n_loc), 2)
              + my_x * n_loc)
        counts = jnp.sum(jnp.where(btm[:, :, None] == pg, 1.0, 0.0),
                         axis=1)
        ii = lax.broadcasted_iota(jnp.int32, (n_loc, n_keys), 0)
        jj = lax.broadcasted_iota(jnp.int32, (n_loc, n_keys), 1)
        expand = jnp.where(jj // bs == ii, 1.0, 0.0)
        w8 = lax.dot_general(counts, expand, (((1,), (0,)), ((), ())),
                             preferred_element_type=jnp.float32)
        w = jnp.broadcast_to(w8[:, None, :], (B, h, n_keys)).reshape(R, n_keys)

        q2 = q_ref[...]
        rblk = lax.broadcasted_iota(jnp.int32, (R, C), 0) % h
        cblk = lax.broadcasted_iota(jnp.int32, (R, C), 1) // d
        bd_mask = rblk == cblk
        qpad = jnp.where(bd_mask, jnp.concatenate([q2] * h, axis=1), 0.0)

        s = lax.dot_general(qpad, k_ref[...], (((1,), (1,)), ((), ())),
                            preferred_element_type=jnp.float32) * SCALE
        s = jnp.where(w > 0, s, -1e30)
        m = jnp.max(s, axis=1, keepdims=True)
        p = jnp.exp(s - m) * w
        den = jnp.sum(p, axis=1, keepdims=True)
        o_full = lax.dot_general(p, v_ref[...], (((1,), (0,)), ((), ())),
                                 preferred_element_type=jnp.float32)
        o_full = jnp.where(bd_mask, o_full, 0.0)
        o_loc = o_full[:, :d]
        for g in range(1, h):
            o_loc = o_loc + o_full[:, g * d:(g + 1) * d]

        send_buf[:, :d] = o_loc
        send_buf[:, d:d + 1] = m
        send_buf[:, d + 1:d + 2] = den

        barrier = pltpu.get_barrier_semaphore()
        pl.semaphore_signal(barrier, inc=1, device_id=peer,
                            device_id_type=pl.DeviceIdType.MESH)
        pl.semaphore_wait(barrier, 1)

        rdma = pltpu.make_async_remote_copy(
            src_ref=send_buf, dst_ref=recv_buf,
            send_sem=send_sem, recv_sem=recv_sem,
            device_id=peer, device_id_type=pl.DeviceIdType.MESH)
        rdma.start()
        rdma.wait()

        o_rem = recv_buf[:, :d]
        m_rem = recv_buf[:, d:d + 1]
        den_rem = recv_buf[:, d + 1:d + 2]
        mm = jnp.maximum(m, m_rem)
        a_loc = jnp.exp(m - mm)
        a_rem = jnp.exp(m_rem - mm)
        num = o_loc * a_loc + o_rem * a_rem
        dd = den * a_loc + den_rem * a_rem
        out_ref[...] = (num / dd).reshape(B, 1, h, d)

    return pl.pallas_call(
        body,
        out_shape=jax.ShapeDtypeStruct(Q.shape, jnp.float32),
        in_specs=[pl.BlockSpec(memory_space=pltpu.VMEM)] * 5,
        out_specs=pl.BlockSpec(memory_space=pltpu.VMEM),
        scratch_shapes=[
            pltpu.VMEM((R, 2 * d), jnp.float32),
            pltpu.VMEM((R, 2 * d), jnp.float32),
            pltpu.SemaphoreType.DMA,
            pltpu.SemaphoreType.DMA,
        ],
        compiler_params=pltpu.CompilerParams(collective_id=0),
    )(Q.reshape(R, d),
      K.reshape(n_keys, C),
      V.reshape(n_keys, C),
      bt,
      lens.reshape(B, 1))
